# baseline (speedup 1.0000x reference)
import functools

import jax
import jax.numpy as jnp
from jax.experimental import pallas as pl
from jax.experimental.pallas import tpu as pltpu

B, C, H, W = 8, 384, 224, 224
HID, E, K = 96, 64, 2
HWTOT = H * W
P = 2
BLK = 512
J = HWTOT // (P * BLK)


def _pool_body(x_ref, out_ref, acc_ref):
    j = pl.program_id(1)

    @pl.when(j == 0)
    def _init():
        acc_ref[...] = jnp.zeros_like(acc_ref)

    acc_ref[...] += jnp.sum(x_ref[...], axis=2)

    @pl.when(j == J - 1)
    def _done():
        out_ref[0] = acc_ref[...]


def _head_body(part_ref, w1_ref, b1_ref, w2_ref, b2_ref,
               idx_ref, val_ref, probs_ref):
    h = jnp.sum(part_ref[...], axis=0) * (1.0 / HWTOT)
    hid = jax.lax.dot_general(h, w1_ref[...],
                              (((1,), (1,)), ((), ())),
                              preferred_element_type=jnp.float32)
    hid = jnp.maximum(hid + b1_ref[...], 0.0)
    logits = jax.lax.dot_general(hid, w2_ref[...],
                                 (((1,), (1,)), ((), ())),
                                 preferred_element_type=jnp.float32)
    logits = logits + b2_ref[...]
    m = jnp.max(logits, axis=1, keepdims=True)
    e = jnp.exp(logits - m)
    probs = e / jnp.sum(e, axis=1, keepdims=True)
    probs_ref[...] = probs

    iota = jax.lax.broadcasted_iota(jnp.int32, (B, E), 1)
    m1 = jnp.max(probs, axis=1, keepdims=True)
    i1 = jnp.min(jnp.where(probs == m1, iota, E), axis=1, keepdims=True)
    masked = jnp.where(iota == i1, -jnp.inf, probs)
    m2 = jnp.max(masked, axis=1, keepdims=True)
    i2 = jnp.min(jnp.where(masked == m2, iota, E), axis=1, keepdims=True)
    val_ref[...] = jnp.concatenate([m1, m2], axis=1)
    idx_ref[...] = jnp.concatenate([i1, i2], axis=1)


@jax.jit
def kernel(x, W1, b1, W2, b2):
    x3 = x.reshape(B, C, HWTOT)

    partials = pl.pallas_call(
        _pool_body,
        grid=(P, J),
        in_specs=[pl.BlockSpec((B, C, BLK), lambda p, j: (0, 0, p * J + j))],
        out_specs=pl.BlockSpec((1, B, C), lambda p, j: (p, 0, 0)),
        out_shape=jax.ShapeDtypeStruct((P, B, C), jnp.float32),
        scratch_shapes=[pltpu.VMEM((B, C), jnp.float32)],
        compiler_params=pltpu.CompilerParams(
            dimension_semantics=("parallel", "arbitrary")),
    )(x3)

    idx, val, probs = pl.pallas_call(
        _head_body,
        in_specs=[pl.BlockSpec(partials.shape, lambda: (0, 0, 0)),
                  pl.BlockSpec(W1.shape, lambda: (0, 0)),
                  pl.BlockSpec((1, HID), lambda: (0, 0)),
                  pl.BlockSpec(W2.shape, lambda: (0, 0)),
                  pl.BlockSpec((1, E), lambda: (0, 0))],
        out_specs=[pl.BlockSpec((B, K), lambda: (0, 0)),
                   pl.BlockSpec((B, K), lambda: (0, 0)),
                   pl.BlockSpec((B, E), lambda: (0, 0))],
        out_shape=[jax.ShapeDtypeStruct((B, K), jnp.int32),
                   jax.ShapeDtypeStruct((B, K), jnp.float32),
                   jax.ShapeDtypeStruct((B, E), jnp.float32)],
    )(partials, W1, b1.reshape(1, HID), W2, b2.reshape(1, E))

    return (idx, val, probs)
